# baseline (speedup 1.0000x reference)
import functools

import jax
import jax.numpy as jnp
import numpy as np
from jax import lax
from jax.experimental import pallas as pl
from jax.experimental.pallas import tpu as pltpu
from jax.experimental.pallas import tpu_sc as plsc

N = 8388608
TOTAL_BINS = 64 * 64 * 64

NUM_CORES = 2
NUM_SUBCORES = 16
NW = NUM_CORES * NUM_SUBCORES
PTS_PER_W = N // NW
CHUNK = 2048
NCH = PTS_PER_W // CHUNK
VL = 16
BINS_PER_TILE = TOTAL_BINS // NUM_SUBCORES

SCALE = np.float32(64.0 / 6.0)
BIAS = np.float32(32.0 + 1024.0)


def _sc_body(vals_hbm, w_hbm, hist_out, oob_out,
             vals_v, w_v, idx_v, wv_v, oob_v, hist_sp):
    cid = lax.axis_index("c")
    sid = lax.axis_index("s")
    wid = sid * NUM_CORES + cid

    if True:
        zeros16 = jnp.zeros((VL,), jnp.float32)

        def zero_body(k, _):
            wv_v[pl.ds(k * VL, VL)] = zeros16
            return 0

        lax.fori_loop(0, CHUNK // VL, zero_body, 0)
        for q in range(BINS_PER_TILE // CHUNK):
            pltpu.sync_copy(wv_v, hist_sp.at[pl.ds(sid * BINS_PER_TILE + q * CHUNK, CHUNK)])
        plsc.subcore_barrier()

        lanes = lax.iota(jnp.int32, VL)
        lanes3 = lanes * 3
        spread0 = (lanes + wid * 8191) & (TOTAL_BINS - 1)

        def chunk_body(g, oob_acc):
            base = wid * PTS_PER_W + g * CHUNK
            pltpu.sync_copy(vals_hbm.at[pl.ds(base * 3, CHUNK * 3)], vals_v)
            pltpu.sync_copy(w_hbm.at[pl.ds(base, CHUNK)], w_v)

            def pt_body(j, acc):
                ix = lanes3 + j * (3 * VL)
                vx = plsc.load_gather(vals_v, [ix])
                vy = plsc.load_gather(vals_v, [ix + 1])
                vz = plsc.load_gather(vals_v, [ix + 2])
                bx = (vx * SCALE + BIAS).astype(jnp.int32) - 1024
                by = (vy * SCALE + BIAS).astype(jnp.int32) - 1024
                bz = (vz * SCALE + BIAS).astype(jnp.int32) - 1024
                okx = plsc.bitcast(bx, jnp.uint32) < 64
                oky = plsc.bitcast(by, jnp.uint32) < 64
                okz = plsc.bitcast(bz, jnp.uint32) < 64
                ok = okx & oky & okz
                flat = (bx << 12) + (by << 6) + bz
                spread = (spread0 + (g * CHUNK + j * VL)) & (TOTAL_BINS - 1)
                flat = jnp.where(ok, flat, spread)
                w = w_v[pl.ds(j * VL, VL)]
                wv = jnp.where(ok, w, jnp.float32(0.0))
                idx_v[pl.ds(j * VL, VL)] = flat
                wv_v[pl.ds(j * VL, VL)] = wv
                return acc + (w - wv)

            oob_acc = lax.fori_loop(0, CHUNK // VL, pt_body, oob_acc)
            pltpu.sync_copy(wv_v, hist_sp.at[idx_v], add=True)
            return oob_acc

        oob_acc = lax.fori_loop(0, NCH, chunk_body, jnp.zeros((VL,), jnp.float32))
        plsc.subcore_barrier()

        pltpu.sync_copy(
            hist_sp.at[pl.ds(sid * BINS_PER_TILE, BINS_PER_TILE)],
            hist_out.at[cid, pl.ds(sid * BINS_PER_TILE, BINS_PER_TILE)],
        )
        oob_v[...] = oob_acc
        pltpu.sync_copy(oob_v, oob_out.at[cid, sid])


@functools.cache
def _build_sc_hist():
    return pl.kernel(
        _sc_body,
        out_type=(
            jax.ShapeDtypeStruct((NUM_CORES, TOTAL_BINS), jnp.float32),
            jax.ShapeDtypeStruct((NUM_CORES, NUM_SUBCORES, VL), jnp.float32),
        ),
        mesh=plsc.VectorSubcoreMesh(
            core_axis_name="c", subcore_axis_name="s",
            num_cores=NUM_CORES, num_subcores=NUM_SUBCORES,
        ),
        scratch_types=[
            pltpu.VMEM((CHUNK * 3,), jnp.float32),
            pltpu.VMEM((CHUNK,), jnp.float32),
            pltpu.VMEM((CHUNK,), jnp.int32),
            pltpu.VMEM((CHUNK,), jnp.float32),
            pltpu.VMEM((VL,), jnp.float32),
            pltpu.VMEM_SHARED((TOTAL_BINS,), jnp.float32),
        ],
        compiler_params=pltpu.CompilerParams(needs_layout_passes=False),
    )


def _combine_body(hp_ref, oob_ref, hist_ref, oob_out_ref):
    hist_ref[...] = hp_ref[0] + hp_ref[1]
    oob_out_ref[...] = jnp.sum(oob_ref[...])[None, None]


def kernel(values, weights):
    hist_parts, oob_parts = _build_sc_hist()(values.reshape(-1), weights)
    hist2, oob11 = pl.pallas_call(
        _combine_body,
        out_shape=(
            jax.ShapeDtypeStruct((TOTAL_BINS // 128, 128), jnp.float32),
            jax.ShapeDtypeStruct((1, 1), jnp.float32),
        ),
    )(hist_parts.reshape(NUM_CORES, TOTAL_BINS // 128, 128),
      oob_parts.reshape(4, 128))
    return hist2.reshape(TOTAL_BINS), oob11[0, 0]


# speedup vs baseline: 1.2349x; 1.2349x over previous
import functools

import jax
import jax.numpy as jnp
import numpy as np
from jax import lax
from jax.experimental import pallas as pl
from jax.experimental.pallas import tpu as pltpu
from jax.experimental.pallas import tpu_sc as plsc

N = 8388608
TOTAL_BINS = 64 * 64 * 64

NUM_CORES = 2
NUM_SUBCORES = 16
NW = NUM_CORES * NUM_SUBCORES
PTS_PER_W = N // NW
CHUNK = 8192
NCH = PTS_PER_W // CHUNK
VL = 16
BINS_PER_TILE = TOTAL_BINS // NUM_SUBCORES

BINB = 4096
NBLK = N // BINB


def _bin_body(v_ref, w_ref, idx_ref, wv_ref, oob_ref):
    pid = pl.program_id(0)
    v = v_ref[...]
    t = v * jnp.float32(64.0 / 6.0) + jnp.float32(32.0 + 1024.0)
    b = t.astype(jnp.int32) - 1024
    okd = lax.bitcast_convert_type(b, jnp.uint32) < jnp.uint32(64)
    stride = 1 << (lax.broadcasted_iota(jnp.int32, (1, 3), 1) * -6 + 12)
    contrib = jnp.where(okd, b * stride, jnp.int32(1 << 22))
    raw = jnp.sum(contrib, axis=1).reshape(BINB // 128, 128)
    ok2 = raw < (1 << 22)
    spread = (
        lax.broadcasted_iota(jnp.int32, (BINB // 128, 128), 0) * 128
        + lax.broadcasted_iota(jnp.int32, (BINB // 128, 128), 1)
        + pid * BINB
    ) & (TOTAL_BINS - 1)
    idx_ref[...] = jnp.where(ok2, raw, spread)
    w = w_ref[...]
    wv = jnp.where(ok2, w, jnp.float32(0.0))
    wv_ref[...] = wv
    oob_ref[...] = jnp.sum(w - wv)[None, None, None]


@functools.cache
def _build_binning():
    return pl.pallas_call(
        _bin_body,
        grid=(NBLK,),
        in_specs=[
            pl.BlockSpec((BINB, 3), lambda i: (i, 0)),
            pl.BlockSpec((BINB // 128, 128), lambda i: (i, 0)),
        ],
        out_specs=[
            pl.BlockSpec((BINB // 128, 128), lambda i: (i, 0)),
            pl.BlockSpec((BINB // 128, 128), lambda i: (i, 0)),
            pl.BlockSpec((1, 1, 1), lambda i: (i, 0, 0)),
        ],
        out_shape=(
            jax.ShapeDtypeStruct((N // 128, 128), jnp.int32),
            jax.ShapeDtypeStruct((N // 128, 128), jnp.float32),
            jax.ShapeDtypeStruct((NBLK, 1, 1), jnp.float32),
        ),
    )


def _sc_body(idx_hbm, wv_hbm, hist_out, idx_v, wv_v, hist_sp):
    cid = lax.axis_index("c")
    sid = lax.axis_index("s")
    wid = sid * NUM_CORES + cid

    zeros16 = jnp.zeros((VL,), jnp.float32)

    def zero_body(k, _):
        wv_v[pl.ds(k * VL, VL)] = zeros16
        return 0

    lax.fori_loop(0, CHUNK // VL, zero_body, 0)
    for q in range(BINS_PER_TILE // CHUNK):
        pltpu.sync_copy(
            wv_v, hist_sp.at[pl.ds(sid * BINS_PER_TILE + q * CHUNK, CHUNK)]
        )
    plsc.subcore_barrier()

    def chunk_body(g, carry):
        base = wid * PTS_PER_W + g * CHUNK
        pltpu.sync_copy(idx_hbm.at[pl.ds(base, CHUNK)], idx_v)
        pltpu.sync_copy(wv_hbm.at[pl.ds(base, CHUNK)], wv_v)
        pltpu.sync_copy(wv_v, hist_sp.at[idx_v], add=True)
        return carry

    lax.fori_loop(0, NCH, chunk_body, 0)
    plsc.subcore_barrier()

    pltpu.sync_copy(
        hist_sp.at[pl.ds(sid * BINS_PER_TILE, BINS_PER_TILE)],
        hist_out.at[cid, pl.ds(sid * BINS_PER_TILE, BINS_PER_TILE)],
    )


@functools.cache
def _build_sc_scatter():
    return pl.kernel(
        _sc_body,
        out_type=jax.ShapeDtypeStruct((NUM_CORES, TOTAL_BINS), jnp.float32),
        mesh=plsc.VectorSubcoreMesh(
            core_axis_name="c", subcore_axis_name="s",
            num_cores=NUM_CORES, num_subcores=NUM_SUBCORES,
        ),
        scratch_types=[
            pltpu.VMEM((CHUNK,), jnp.int32),
            pltpu.VMEM((CHUNK,), jnp.float32),
            pltpu.VMEM_SHARED((TOTAL_BINS,), jnp.float32),
        ],
        compiler_params=pltpu.CompilerParams(needs_layout_passes=False),
    )


def _combine_body(hp_ref, oob_ref, hist_ref, oob_out_ref):
    hist_ref[...] = hp_ref[0] + hp_ref[1]
    oob_out_ref[...] = jnp.sum(oob_ref[...])[None, None]


def kernel(values, weights):
    w2 = weights.reshape(N // 128, 128)
    idx, wv, oob_parts = _build_binning()(values, w2)
    hist_parts = _build_sc_scatter()(idx.reshape(N), wv.reshape(N))
    hist2, oob11 = pl.pallas_call(
        _combine_body,
        out_shape=(
            jax.ShapeDtypeStruct((TOTAL_BINS // 128, 128), jnp.float32),
            jax.ShapeDtypeStruct((1, 1), jnp.float32),
        ),
    )(hist_parts.reshape(NUM_CORES, TOTAL_BINS // 128, 128),
      oob_parts.reshape(NBLK // 128, 128))
    return hist2.reshape(TOTAL_BINS), oob11[0, 0]


# speedup vs baseline: 2.5329x; 2.0512x over previous
import functools

import jax
import jax.numpy as jnp
import numpy as np
from jax import lax
from jax.experimental import pallas as pl
from jax.experimental.pallas import tpu as pltpu
from jax.experimental.pallas import tpu_sc as plsc

N = 8388608
TOTAL_BINS = 64 * 64 * 64

NUM_CORES = 2
NUM_SUBCORES = 16
NW = NUM_CORES * NUM_SUBCORES
PTS_PER_W = N // NW
CHUNK = 512
NCH = PTS_PER_W // CHUNK
VL = 16
BINS_PER_TILE = TOTAL_BINS // NUM_SUBCORES

SCALE = np.float32(64.0 / 6.0)
BIAS = np.float32(32.0 + 1024.0)


def _sc_body(vals_hbm, w_hbm, hist_out, oob_out,
             vals_v, w_v, idx_v, wv_v, oob_v, hist_sp):
    cid = lax.axis_index("c")
    sid = lax.axis_index("s")
    wid = sid * NUM_CORES + cid

    zeros16 = jnp.zeros((VL,), jnp.float32)
    lanes = lax.iota(jnp.int32, VL)

    def zero_body(k, _):
        wv_v[pl.ds(k * VL, VL)] = zeros16
        return 0

    lax.fori_loop(0, CHUNK // VL, zero_body, 0)
    for q in range(BINS_PER_TILE // CHUNK):
        pltpu.sync_copy(
            wv_v, hist_sp.at[pl.ds(sid * BINS_PER_TILE + q * CHUNK, CHUNK)]
        )
    plsc.subcore_barrier()

    dim0 = jnp.zeros((VL,), jnp.int32)
    dim1 = dim0 + 1
    dim2 = dim0 + 2
    spread0 = (lanes + wid * 8191) & (TOTAL_BINS - 1)

    def chunk_body(g, oob_acc):
        base = wid * PTS_PER_W + g * CHUNK
        pltpu.sync_copy(vals_hbm.at[pl.ds(base, CHUNK)], vals_v)
        pltpu.sync_copy(w_hbm.at[pl.ds(base, CHUNK)], w_v)

        def pt_body(j, acc):
            pt = lanes + j * VL
            vx = plsc.load_gather(vals_v, [pt, dim0])
            vy = plsc.load_gather(vals_v, [pt, dim1])
            vz = plsc.load_gather(vals_v, [pt, dim2])
            bx = (vx * SCALE + BIAS).astype(jnp.int32) - 1024
            by = (vy * SCALE + BIAS).astype(jnp.int32) - 1024
            bz = (vz * SCALE + BIAS).astype(jnp.int32) - 1024
            okx = plsc.bitcast(bx, jnp.uint32) < 64
            oky = plsc.bitcast(by, jnp.uint32) < 64
            okz = plsc.bitcast(bz, jnp.uint32) < 64
            ok = okx & oky & okz
            flat = (bx << 12) + (by << 6) + bz
            spread = (spread0 + (g * CHUNK + j * VL)) & (TOTAL_BINS - 1)
            flat = jnp.where(ok, flat, spread)
            w = w_v[pl.ds(j * VL, VL)]
            wv = jnp.where(ok, w, jnp.float32(0.0))
            idx_v[pl.ds(j * VL, VL)] = flat
            wv_v[pl.ds(j * VL, VL)] = wv
            return acc + (w - wv)

        oob_acc = lax.fori_loop(0, CHUNK // VL, pt_body, oob_acc)
        pltpu.sync_copy(wv_v, hist_sp.at[idx_v], add=True)
        return oob_acc

    oob_acc = lax.fori_loop(0, NCH, chunk_body, jnp.zeros((VL,), jnp.float32))
    plsc.subcore_barrier()

    pltpu.sync_copy(
        hist_sp.at[pl.ds(sid * BINS_PER_TILE, BINS_PER_TILE)],
        hist_out.at[cid, pl.ds(sid * BINS_PER_TILE, BINS_PER_TILE)],
    )
    oob_v[...] = oob_acc
    pltpu.sync_copy(oob_v, oob_out.at[cid, sid])


@functools.cache
def _build_sc_hist():
    return pl.kernel(
        _sc_body,
        out_type=(
            jax.ShapeDtypeStruct((NUM_CORES, TOTAL_BINS), jnp.float32),
            jax.ShapeDtypeStruct((NUM_CORES, NUM_SUBCORES, VL), jnp.float32),
        ),
        mesh=plsc.VectorSubcoreMesh(
            core_axis_name="c", subcore_axis_name="s",
            num_cores=NUM_CORES, num_subcores=NUM_SUBCORES,
        ),
        scratch_types=[
            pltpu.VMEM((CHUNK, 3), jnp.float32),
            pltpu.VMEM((CHUNK,), jnp.float32),
            pltpu.VMEM((CHUNK,), jnp.int32),
            pltpu.VMEM((CHUNK,), jnp.float32),
            pltpu.VMEM((VL,), jnp.float32),
            pltpu.VMEM_SHARED((TOTAL_BINS,), jnp.float32),
        ],
        compiler_params=pltpu.CompilerParams(needs_layout_passes=False),
    )


def _combine_body(hp_ref, oob_ref, hist_ref, oob_out_ref):
    hist_ref[...] = hp_ref[0] + hp_ref[1]
    oob_out_ref[...] = jnp.sum(oob_ref[...])[None, None]


def kernel(values, weights):
    hist_parts, oob_parts = _build_sc_hist()(values, weights)
    hist2, oob11 = pl.pallas_call(
        _combine_body,
        out_shape=(
            jax.ShapeDtypeStruct((TOTAL_BINS // 128, 128), jnp.float32),
            jax.ShapeDtypeStruct((1, 1), jnp.float32),
        ),
    )(hist_parts.reshape(NUM_CORES, TOTAL_BINS // 128, 128),
      oob_parts.reshape(4, 128))
    return hist2.reshape(TOTAL_BINS), oob11[0, 0]


# speedup vs baseline: 2.9967x; 1.1831x over previous
import functools

import jax
import jax.numpy as jnp
import numpy as np
from jax import lax
from jax.experimental import pallas as pl
from jax.experimental.pallas import tpu as pltpu
from jax.experimental.pallas import tpu_sc as plsc

N = 8388608
TOTAL_BINS = 64 * 64 * 64

NUM_CORES = 2
NUM_SUBCORES = 16
NW = NUM_CORES * NUM_SUBCORES
PTS_PER_W = N // NW
CHUNK = 256
NCH = PTS_PER_W // CHUNK
VL = 16
BINS_PER_TILE = TOTAL_BINS // NUM_SUBCORES

SCALE = np.float32(64.0 / 6.0)
BIAS = np.float32(32.0 + 1024.0)


def _sc_body(vals_hbm, w_hbm, hist_out, oob_out,
             vals_v0, vals_v1, w_v0, w_v1, idx_v0, idx_v1, wv_v0, wv_v1,
             oob_v, hist_sp, sem0, sem1):
    cid = lax.axis_index("c")
    sid = lax.axis_index("s")
    wid = sid * NUM_CORES + cid
    vals_b = (vals_v0, vals_v1)
    w_b = (w_v0, w_v1)
    idx_b = (idx_v0, idx_v1)
    wv_b = (wv_v0, wv_v1)
    sem_b = (sem0, sem1)

    zeros16 = jnp.zeros((VL,), jnp.float32)
    lanes = lax.iota(jnp.int32, VL)

    def zero_body(k, _):
        wv_v0[pl.ds(k * VL, VL)] = zeros16
        return 0

    lax.fori_loop(0, CHUNK // VL, zero_body, 0)
    for q in range(BINS_PER_TILE // CHUNK):
        pltpu.sync_copy(
            wv_v0, hist_sp.at[pl.ds(sid * BINS_PER_TILE + q * CHUNK, CHUNK)]
        )
    plsc.subcore_barrier()

    dim0 = jnp.zeros((VL,), jnp.int32)
    dim1 = dim0 + 1
    dim2 = dim0 + 2
    spread0 = (lanes + wid * 8191) & (TOTAL_BINS - 1)
    w_base = wid * PTS_PER_W

    def fire(win, b):
        base = w_base + jnp.minimum(win, NCH - 1) * CHUNK
        pltpu.async_copy(vals_hbm.at[pl.ds(base, CHUNK)], vals_b[b], sem_b[b])
        pltpu.async_copy(w_hbm.at[pl.ds(base, CHUNK)], w_b[b], sem_b[b])

    def drain(b):
        pltpu.make_async_copy(
            vals_hbm.at[pl.ds(w_base, CHUNK)], vals_b[b], sem_b[b]
        ).wait()
        pltpu.make_async_copy(
            w_hbm.at[pl.ds(w_base, CHUNK)], w_b[b], sem_b[b]
        ).wait()

    def window(win, b, oob_acc):
        vals_v, w_v, idx_v, wv_v = vals_b[b], w_b[b], idx_b[b], wv_b[b]

        def pt_body(j, acc):
            pt = lanes + j * VL
            vx = plsc.load_gather(vals_v, [pt, dim0])
            vy = plsc.load_gather(vals_v, [pt, dim1])
            vz = plsc.load_gather(vals_v, [pt, dim2])
            bx = (vx * SCALE + BIAS).astype(jnp.int32) - 1024
            by = (vy * SCALE + BIAS).astype(jnp.int32) - 1024
            bz = (vz * SCALE + BIAS).astype(jnp.int32) - 1024
            okx = plsc.bitcast(bx, jnp.uint32) < 64
            oky = plsc.bitcast(by, jnp.uint32) < 64
            okz = plsc.bitcast(bz, jnp.uint32) < 64
            ok = okx & oky & okz
            flat = (bx << 12) + (by << 6) + bz
            spread = (spread0 + (win * CHUNK + j * VL)) & (TOTAL_BINS - 1)
            flat = jnp.where(ok, flat, spread)
            w = w_v[pl.ds(j * VL, VL)]
            wv = jnp.where(ok, w, jnp.float32(0.0))
            idx_v[pl.ds(j * VL, VL)] = flat
            wv_v[pl.ds(j * VL, VL)] = wv
            return acc + (w - wv)

        oob_acc = lax.fori_loop(0, CHUNK // VL, pt_body, oob_acc)
        pltpu.sync_copy(wv_v, hist_sp.at[idx_v], add=True)
        return oob_acc

    fire(0, 0)

    def pair_body(g2, oob_acc):
        win = g2 * 2
        fire(win + 1, 1)
        drain(0)
        oob_acc = window(win, 0, oob_acc)
        fire(win + 2, 0)
        drain(1)
        oob_acc = window(win + 1, 1, oob_acc)
        return oob_acc

    oob_acc = lax.fori_loop(0, NCH // 2, pair_body,
                            jnp.zeros((VL,), jnp.float32))
    drain(0)
    plsc.subcore_barrier()

    pltpu.sync_copy(
        hist_sp.at[pl.ds(sid * BINS_PER_TILE, BINS_PER_TILE)],
        hist_out.at[cid, pl.ds(sid * BINS_PER_TILE, BINS_PER_TILE)],
    )
    oob_v[...] = oob_acc
    pltpu.sync_copy(oob_v, oob_out.at[cid, sid])


@functools.cache
def _build_sc_hist():
    return pl.kernel(
        _sc_body,
        out_type=(
            jax.ShapeDtypeStruct((NUM_CORES, TOTAL_BINS), jnp.float32),
            jax.ShapeDtypeStruct((NUM_CORES, NUM_SUBCORES, VL), jnp.float32),
        ),
        mesh=plsc.VectorSubcoreMesh(
            core_axis_name="c", subcore_axis_name="s",
            num_cores=NUM_CORES, num_subcores=NUM_SUBCORES,
        ),
        scratch_types=[
            pltpu.VMEM((CHUNK, 3), jnp.float32),
            pltpu.VMEM((CHUNK, 3), jnp.float32),
            pltpu.VMEM((CHUNK,), jnp.float32),
            pltpu.VMEM((CHUNK,), jnp.float32),
            pltpu.VMEM((CHUNK,), jnp.int32),
            pltpu.VMEM((CHUNK,), jnp.int32),
            pltpu.VMEM((CHUNK,), jnp.float32),
            pltpu.VMEM((CHUNK,), jnp.float32),
            pltpu.VMEM((VL,), jnp.float32),
            pltpu.VMEM_SHARED((TOTAL_BINS,), jnp.float32),
            pltpu.SemaphoreType.DMA,
            pltpu.SemaphoreType.DMA,
        ],
        compiler_params=pltpu.CompilerParams(needs_layout_passes=False),
    )


def _combine_body(hp_ref, oob_ref, hist_ref, oob_out_ref):
    hist_ref[...] = hp_ref[0] + hp_ref[1]
    oob_out_ref[...] = jnp.sum(oob_ref[...])[None, None]


def kernel(values, weights):
    hist_parts, oob_parts = _build_sc_hist()(values, weights)
    hist2, oob11 = pl.pallas_call(
        _combine_body,
        out_shape=(
            jax.ShapeDtypeStruct((TOTAL_BINS // 128, 128), jnp.float32),
            jax.ShapeDtypeStruct((1, 1), jnp.float32),
        ),
    )(hist_parts.reshape(NUM_CORES, TOTAL_BINS // 128, 128),
      oob_parts.reshape(4, 128))
    return hist2.reshape(TOTAL_BINS), oob11[0, 0]
